# baseline (speedup 1.0000x reference)
import jax
import jax.numpy as jnp
from jax.experimental import pallas as pl
from jax.experimental.pallas import tpu as pltpu

_ANCHORS = (
    (1.3221, 1.73145),
    (3.19275, 4.00944),
    (5.05587, 8.09892),
    (9.47112, 4.84053),
    (11.2364, 10.0071),
)
_NUM_CLASSES = 20
_LAMBDA_COORD = 5.0
_LAMBDA_OBJ = 1.0
_LAMBDA_NOOBJ = 0.5
_LAMBDA_CLS = 1.0
_B, _H, _W = 64, 19, 19
_A = len(_ANCHORS)
_HW = _H * _W
_BB = 8


def _loss_body(pred_ref, tgt_ref, box_ref, conf_ref, noobj_ref, cls_ref):
    tgt = jnp.swapaxes(tgt_ref[...], 1, 2)

    gconf = tgt[:, 20, :]
    gx = tgt[:, 21, :]
    gy = tgt[:, 22, :]
    gw = tgt[:, 23, :]
    gh = tgt[:, 24, :]

    b2x1 = gx - gw / 2.0
    b2y1 = gy - gh / 2.0
    b2x2 = gx + gw / 2.0
    b2y2 = gy + gh / 2.0
    a2 = (b2x2 - b2x1) * (b2y2 - b2y1)

    best_iou = None
    best_idx = None
    sel_conf = sel_x = sel_y = sel_w = sel_h = None

    for a in range(_A):
        base = a * (5 + _NUM_CLASSES)
        aw, ah = _ANCHORS[a]
        tconf = jax.nn.sigmoid(pred_ref[:, base + 20, :])
        px = jax.nn.sigmoid(pred_ref[:, base + 21, :])
        py = jax.nn.sigmoid(pred_ref[:, base + 22, :])
        pw = jnp.exp(pred_ref[:, base + 23, :]) * aw
        ph = jnp.exp(pred_ref[:, base + 24, :]) * ah

        b1x1 = px - pw / 2.0
        b1y1 = py - ph / 2.0
        b1x2 = px + pw / 2.0
        b1y2 = py + ph / 2.0
        ix1 = jnp.maximum(b1x1, b2x1)
        iy1 = jnp.maximum(b1y1, b2y1)
        ix2 = jnp.minimum(b1x2, b2x2)
        iy2 = jnp.minimum(b1y2, b2y2)
        iw = jnp.maximum(ix2 - ix1, 0.0)
        ih = jnp.maximum(iy2 - iy1, 0.0)
        inter = iw * ih
        a1 = (b1x2 - b1x1) * (b1y2 - b1y1)
        union = a1 + a2 - inter
        iou = inter / (union + 1e-10)

        if a == 0:
            best_iou = iou
            best_idx = jnp.zeros_like(iou)
            sel_conf, sel_x, sel_y, sel_w, sel_h = tconf, px, py, pw, ph
        else:
            upd = iou > best_iou
            best_iou = jnp.where(upd, iou, best_iou)
            best_idx = jnp.where(upd, float(a), best_idx)
            sel_conf = jnp.where(upd, tconf, sel_conf)
            sel_x = jnp.where(upd, px, sel_x)
            sel_y = jnp.where(upd, py, sel_y)
            sel_w = jnp.where(upd, pw, sel_w)
            sel_h = jnp.where(upd, ph, sel_h)

    masks = [(best_idx == float(a)).astype(jnp.float32) for a in range(_A)]
    sel_logits = []
    for c in range(_NUM_CLASSES):
        sl = masks[0] * pred_ref[:, c, :]
        for a in range(1, _A):
            sl = sl + masks[a] * pred_ref[:, a * (5 + _NUM_CLASSES) + c, :]
        sel_logits.append(sl)

    m = sel_logits[0]
    for c in range(1, _NUM_CLASSES):
        m = jnp.maximum(m, sel_logits[c])
    s = jnp.exp(sel_logits[0] - m)
    pick = tgt[:, 0, :] * sel_logits[0]
    for c in range(1, _NUM_CLASSES):
        s = s + jnp.exp(sel_logits[c] - m)
        pick = pick + tgt[:, c, :] * sel_logits[c]
    lse = m + jnp.log(s)

    obj = (gconf != 0.0).astype(jnp.float32)
    noobj = ((1.0 - gconf) != 0.0).astype(jnp.float32)

    box_p = jnp.sum(
        obj
        * (
            (sel_x - gx) ** 2
            + (sel_y - gy) ** 2
            + (sel_w - gw) ** 2
            + (sel_h - gh) ** 2
        )
    ) * (_LAMBDA_COORD / _B)
    conf_p = jnp.sum(obj * (sel_conf - gconf) ** 2) * (_LAMBDA_OBJ / _B)
    noobj_p = jnp.sum(noobj * sel_conf**2) * (_LAMBDA_NOOBJ / _B)
    cls_p = jnp.sum(obj * (lse - pick)) * (_LAMBDA_CLS / _B)

    @pl.when(pl.program_id(0) == 0)
    def _init():
        box_ref[0, 0] = 0.0
        conf_ref[0, 0] = 0.0
        noobj_ref[0, 0] = 0.0
        cls_ref[0, 0] = 0.0

    box_ref[0, 0] += box_p
    conf_ref[0, 0] += conf_p
    noobj_ref[0, 0] += noobj_p
    cls_ref[0, 0] += cls_p


def kernel(prediction, target):
    pred = prediction.reshape(_B, _A * (5 + _NUM_CLASSES), _HW)
    tgt = target.reshape(_B, _HW, 25)
    scalar = jax.ShapeDtypeStruct((1, 1), jnp.float32)
    outs = pl.pallas_call(
        _loss_body,
        grid=(_B // _BB,),
        in_specs=[
            pl.BlockSpec((_BB, _A * (5 + _NUM_CLASSES), _HW), lambda i: (i, 0, 0)),
            pl.BlockSpec((_BB, _HW, 25), lambda i: (i, 0, 0)),
        ],
        out_specs=[pl.BlockSpec(memory_space=pltpu.SMEM)] * 4,
        out_shape=[scalar] * 4,
    )(pred, tgt)
    return tuple(o[0, 0] for o in outs)


# speedup vs baseline: 1.2096x; 1.2096x over previous
import jax
import jax.numpy as jnp
from jax.experimental import pallas as pl
from jax.experimental.pallas import tpu as pltpu

_ANCHORS = (
    (1.3221, 1.73145),
    (3.19275, 4.00944),
    (5.05587, 8.09892),
    (9.47112, 4.84053),
    (11.2364, 10.0071),
)
_NUM_CLASSES = 20
_LAMBDA_COORD = 5.0
_LAMBDA_OBJ = 1.0
_LAMBDA_NOOBJ = 0.5
_LAMBDA_CLS = 1.0
_B, _H, _W = 64, 19, 19
_A = len(_ANCHORS)
_HW = _H * _W
_BB = 8


def _loss_body(pred_ref, tgt_ref, box_ref, conf_ref, noobj_ref, cls_ref):
    tgt = tgt_ref[...]

    gconf = tgt[:, 20, :]
    gx = tgt[:, 21, :]
    gy = tgt[:, 22, :]
    gw = tgt[:, 23, :]
    gh = tgt[:, 24, :]

    b2x1 = gx - gw / 2.0
    b2y1 = gy - gh / 2.0
    b2x2 = gx + gw / 2.0
    b2y2 = gy + gh / 2.0
    a2 = (b2x2 - b2x1) * (b2y2 - b2y1)

    best_iou = None
    best_idx = None
    sel_conf = sel_x = sel_y = sel_w = sel_h = None

    for a in range(_A):
        base = a * (5 + _NUM_CLASSES)
        aw, ah = _ANCHORS[a]
        tconf = jax.nn.sigmoid(pred_ref[:, base + 20, :])
        px = jax.nn.sigmoid(pred_ref[:, base + 21, :])
        py = jax.nn.sigmoid(pred_ref[:, base + 22, :])
        pw = jnp.exp(pred_ref[:, base + 23, :]) * aw
        ph = jnp.exp(pred_ref[:, base + 24, :]) * ah

        b1x1 = px - pw / 2.0
        b1y1 = py - ph / 2.0
        b1x2 = px + pw / 2.0
        b1y2 = py + ph / 2.0
        ix1 = jnp.maximum(b1x1, b2x1)
        iy1 = jnp.maximum(b1y1, b2y1)
        ix2 = jnp.minimum(b1x2, b2x2)
        iy2 = jnp.minimum(b1y2, b2y2)
        iw = jnp.maximum(ix2 - ix1, 0.0)
        ih = jnp.maximum(iy2 - iy1, 0.0)
        inter = iw * ih
        a1 = (b1x2 - b1x1) * (b1y2 - b1y1)
        union = a1 + a2 - inter
        iou = inter / (union + 1e-10)

        if a == 0:
            best_iou = iou
            best_idx = jnp.zeros_like(iou)
            sel_conf, sel_x, sel_y, sel_w, sel_h = tconf, px, py, pw, ph
        else:
            upd = iou > best_iou
            best_iou = jnp.where(upd, iou, best_iou)
            best_idx = jnp.where(upd, float(a), best_idx)
            sel_conf = jnp.where(upd, tconf, sel_conf)
            sel_x = jnp.where(upd, px, sel_x)
            sel_y = jnp.where(upd, py, sel_y)
            sel_w = jnp.where(upd, pw, sel_w)
            sel_h = jnp.where(upd, ph, sel_h)

    masks = [(best_idx == float(a)).astype(jnp.float32) for a in range(_A)]
    sel_logits = []
    for c in range(_NUM_CLASSES):
        sl = masks[0] * pred_ref[:, c, :]
        for a in range(1, _A):
            sl = sl + masks[a] * pred_ref[:, a * (5 + _NUM_CLASSES) + c, :]
        sel_logits.append(sl)

    m = sel_logits[0]
    for c in range(1, _NUM_CLASSES):
        m = jnp.maximum(m, sel_logits[c])
    s = jnp.exp(sel_logits[0] - m)
    pick = tgt[:, 0, :] * sel_logits[0]
    for c in range(1, _NUM_CLASSES):
        s = s + jnp.exp(sel_logits[c] - m)
        pick = pick + tgt[:, c, :] * sel_logits[c]
    lse = m + jnp.log(s)

    obj = (gconf != 0.0).astype(jnp.float32)
    noobj = ((1.0 - gconf) != 0.0).astype(jnp.float32)

    box_p = jnp.sum(
        obj
        * (
            (sel_x - gx) ** 2
            + (sel_y - gy) ** 2
            + (sel_w - gw) ** 2
            + (sel_h - gh) ** 2
        )
    ) * (_LAMBDA_COORD / _B)
    conf_p = jnp.sum(obj * (sel_conf - gconf) ** 2) * (_LAMBDA_OBJ / _B)
    noobj_p = jnp.sum(noobj * sel_conf**2) * (_LAMBDA_NOOBJ / _B)
    cls_p = jnp.sum(obj * (lse - pick)) * (_LAMBDA_CLS / _B)

    @pl.when(pl.program_id(0) == 0)
    def _init():
        box_ref[0, 0] = 0.0
        conf_ref[0, 0] = 0.0
        noobj_ref[0, 0] = 0.0
        cls_ref[0, 0] = 0.0

    box_ref[0, 0] += box_p
    conf_ref[0, 0] += conf_p
    noobj_ref[0, 0] += noobj_p
    cls_ref[0, 0] += cls_p


def kernel(prediction, target):
    pred = prediction.reshape(_B, _A * (5 + _NUM_CLASSES), _HW)
    tgt = jnp.transpose(target.reshape(_B, _HW, 25), (0, 2, 1))
    scalar = jax.ShapeDtypeStruct((1, 1), jnp.float32)
    outs = pl.pallas_call(
        _loss_body,
        grid=(_B // _BB,),
        in_specs=[
            pl.BlockSpec((_BB, _A * (5 + _NUM_CLASSES), _HW), lambda i: (i, 0, 0)),
            pl.BlockSpec((_BB, 25, _HW), lambda i: (i, 0, 0)),
        ],
        out_specs=[pl.BlockSpec(memory_space=pltpu.SMEM)] * 4,
        out_shape=[scalar] * 4,
    )(pred, tgt)
    return tuple(o[0, 0] for o in outs)
